# baseline (speedup 1.0000x reference)
import functools

import jax
import jax.numpy as jnp
from jax import lax
from jax.experimental import pallas as pl
from jax.experimental.pallas import tpu as pltpu
from jax.experimental.pallas import tpu_sc as plsc

_INFO = plsc.get_sparse_core_info()
_NC = _INFO.num_cores
_NS = _INFO.num_subcores
_NW = _NC * _NS

V = 1000000
D = 64
SEQ = 200
BATCH = 4096
NB = BATCH // 128
VFULL = (V // 128) * 128
NJ = VFULL // 128
SROWS = V // 2


def _k1_relayout(table_t, tail_packed):
  mesh = plsc.VectorSubcoreMesh(core_axis_name="c", subcore_axis_name="s")

  @functools.partial(
      pl.kernel,
      mesh=mesh,
      out_type=jax.ShapeDtypeStruct((SROWS, 128), jnp.float32),
      compiler_params=pltpu.CompilerParams(use_tc_tiling_on_sc=True, needs_layout_passes=False),
      scratch_types=[
          pltpu.VMEM((2, 64, 128), jnp.float32),
          pltpu.VMEM((64, 128), jnp.float32),
          pltpu.VMEM((32, 128), jnp.float32),
          pltpu.SemaphoreType.DMA((2,)),
          pltpu.SemaphoreType.DMA,
      ],
  )
  def k1(t_hbm, tail_hbm, s_hbm, ib, ob, tb, gsem, wsem):
    wid = lax.axis_index("s") * _NC + lax.axis_index("c")
    iota = lax.iota(jnp.int32, 16)
    dlanes = [iota + (g * 16) for g in range(4)]

    n_rounds = NJ // _NW + 1

    def in_dma(k, b):
      j = wid + k * _NW
      return pltpu.make_async_copy(
          t_hbm.at[:, pl.ds(j * 128, 128)], ib.at[b], gsem.at[b])

    in_dma(0, 0).start()
    in_dma(1, 1).start()

    def body(k, carry):
      j = wid + k * _NW

      @pl.when(j < NJ)
      def _():
        for b in range(2):
          @pl.when(lax.rem(k, 2) == b)
          def _():
            in_dma(k, b).wait()
            for jj in range(64):
              for p in range(2):
                vcol = jnp.full((16,), 2 * jj + p, jnp.int32)
                for g in range(4):
                  vals = plsc.load_gather(ib.at[b], [dlanes[g], vcol])
                  ob[jj, pl.ds(p * 64 + g * 16, 16)] = vals
            @pl.when(j + 2 * _NW < NJ)
            def _():
              in_dma(k + 2, b).start()

        pltpu.sync_copy(ob, s_hbm.at[pl.ds(j * 64, 64)])
      return carry

    lax.fori_loop(0, n_rounds, body, 0)

    @pl.when(wid == 0)
    def _():
      pltpu.async_copy(tail_hbm, tb, wsem).wait()
      pltpu.sync_copy(tb, s_hbm.at[pl.ds(NJ * 64, 32)])

  return k1(table_t, tail_packed)


def _k2_gather(scratch, idx_t):
  mesh = plsc.VectorSubcoreMesh(core_axis_name="c", subcore_axis_name="s")
  n_items = (SEQ // 8) * NB
  per_w = n_items // _NW

  @functools.partial(
      pl.kernel,
      mesh=mesh,
      out_type=jax.ShapeDtypeStruct((SEQ, 8, NB, 8, 128), jnp.float32),
      compiler_params=pltpu.CompilerParams(use_tc_tiling_on_sc=True, needs_layout_passes=False),
      scratch_types=[
          pltpu.VMEM((8, 128), jnp.int32),
          pltpu.VMEM((8, 128), jnp.int32),
          pltpu.VMEM((2, 128, 128), jnp.float32),
          pltpu.VMEM((2, 8, 8, 128), jnp.float32),
          pltpu.SemaphoreType.DMA,
          pltpu.SemaphoreType.DMA((2,)),
          pltpu.SemaphoreType.DMA((2,)),
      ],
  )
  def k2(s_hbm, i_hbm, o_hbm, idxb, pairb, gat, outb, isem, gsem, wsem):
    wid = lax.axis_index("s") * _NC + lax.axis_index("c")
    iota = lax.iota(jnp.int32, 16)
    tvecs = [iota + (t * 16) for t in range(8)]

    def gather_dma(r, b):
      return pltpu.make_async_copy(
          s_hbm.at[pairb.at[r]], gat.at[b], gsem.at[b])

    def out_wait(b):
      pltpu.make_async_copy(
          outb.at[b], o_hbm.at[0, :, 0], wsem.at[b]).wait()

    def body(m, carry):
      q = wid * per_w + m
      sg = q // NB
      bb = lax.rem(q, NB)

      pltpu.async_copy(
          i_hbm.at[pl.ds(sg * 8, 8), pl.ds(bb * 128, 128)], idxb,
          isem).wait()

      for rr in range(8):
        for g in range(8):
          ii = idxb[rr, pl.ds(g * 16, 16)]
          pairb[rr, pl.ds(g * 16, 16)] = lax.shift_right_logical(ii, 1)

      gather_dma(0, 0).start()
      gather_dma(1, 1).start()

      def rbody(r, rc):
        s_pos = sg * 8 + r

        for b in range(2):
          @pl.when(lax.rem(r, 2) == b)
          def _():
            gather_dma(r, b).wait()

            @pl.when(r >= 2)
            def _():
              out_wait(b)

            for t in range(8):
              par = lax.shift_left(
                  lax.bitwise_and(idxb[r, pl.ds(t * 16, 16)], 1), 6)
              for d in range(64):
                cvec = par + d
                vals = plsc.load_gather(gat.at[b], [tvecs[t], cvec])
                outb[b, d // 8, d % 8, pl.ds(t * 16, 16)] = vals

            pltpu.async_copy(
                outb.at[b], o_hbm.at[s_pos, :, bb], wsem.at[b])

            @pl.when(r + 2 < 8)
            def _():
              gather_dma(r + 2, b).start()
        return rc

      lax.fori_loop(0, 8, rbody, 0)

      out_wait(0)
      out_wait(1)
      return carry

    lax.fori_loop(0, per_w, body, 0)

  return k2(scratch, idx_t)


def kernel(table, indices):
  table_t = table.T
  idx_t = indices.T.astype(jnp.int32)
  tail_packed = table[VFULL:].reshape(32, 128)

  scratch = _k1_relayout(table_t, tail_packed)
  out5 = _k2_gather(scratch, idx_t)
  return out5.transpose(2, 4, 0, 1, 3).reshape(BATCH, SEQ, D)


# speedup vs baseline: 1.8106x; 1.8106x over previous
import functools

import jax
import jax.numpy as jnp
from jax import lax
from jax.experimental import pallas as pl
from jax.experimental.pallas import tpu as pltpu
from jax.experimental.pallas import tpu_sc as plsc

_INFO = plsc.get_sparse_core_info()
_NC = _INFO.num_cores
_NS = _INFO.num_subcores
_NW = _NC * _NS

V = 1000000
D = 64
SEQ = 200
BATCH = 4096
NB = BATCH // 128
VFULL = (V // 128) * 128
NJ = VFULL // 128
SROWS = V // 2


def _k1_relayout(table_t, tail_packed):
  mesh = plsc.VectorSubcoreMesh(core_axis_name="c", subcore_axis_name="s")

  @functools.partial(
      pl.kernel,
      mesh=mesh,
      out_type=jax.ShapeDtypeStruct((SROWS, 128), jnp.float32),
      compiler_params=pltpu.CompilerParams(use_tc_tiling_on_sc=True, needs_layout_passes=False),
      scratch_types=[
          pltpu.VMEM((2, 64, 128), jnp.float32),
          pltpu.VMEM((64, 128), jnp.float32),
          pltpu.VMEM((32, 128), jnp.float32),
          pltpu.SemaphoreType.DMA((2,)),
          pltpu.SemaphoreType.DMA,
      ],
  )
  def k1(t_hbm, tail_hbm, s_hbm, ib, ob, tb, gsem, wsem):
    wid = lax.axis_index("s") * _NC + lax.axis_index("c")
    iota = lax.iota(jnp.int32, 16)
    dlanes = [iota + (g * 16) for g in range(4)]

    n_rounds = NJ // _NW + 1

    def in_dma(k, b):
      j = wid + k * _NW
      return pltpu.make_async_copy(
          t_hbm.at[:, pl.ds(j * 128, 128)], ib.at[b], gsem.at[b])

    in_dma(0, 0).start()
    in_dma(1, 1).start()

    def body(k, carry):
      j = wid + k * _NW

      @pl.when(j < NJ)
      def _():
        for b in range(2):
          @pl.when(lax.rem(k, 2) == b)
          def _():
            in_dma(k, b).wait()

            @plsc.parallel_loop(0, 64, unroll=8)
            def _(jj):
              for p in range(2):
                vcol = jnp.full((16,), 2 * jj + p, jnp.int32)
                for g in range(4):
                  vals = plsc.load_gather(ib.at[b], [dlanes[g], vcol])
                  ob[jj, pl.ds(p * 64 + g * 16, 16)] = vals
            @pl.when(j + 2 * _NW < NJ)
            def _():
              in_dma(k + 2, b).start()

        pltpu.sync_copy(ob, s_hbm.at[pl.ds(j * 64, 64)])
      return carry

    lax.fori_loop(0, n_rounds, body, 0)

    @pl.when(wid == 0)
    def _():
      pltpu.async_copy(tail_hbm, tb, wsem).wait()
      pltpu.sync_copy(tb, s_hbm.at[pl.ds(NJ * 64, 32)])

  return k1(table_t, tail_packed)


def _k2_gather(scratch, idx_t):
  mesh = plsc.VectorSubcoreMesh(core_axis_name="c", subcore_axis_name="s")
  n_items = (SEQ // 8) * NB
  per_w = n_items // _NW

  @functools.partial(
      pl.kernel,
      mesh=mesh,
      out_type=jax.ShapeDtypeStruct((SEQ, 8, NB, 8, 128), jnp.float32),
      compiler_params=pltpu.CompilerParams(use_tc_tiling_on_sc=True, needs_layout_passes=False),
      scratch_types=[
          pltpu.VMEM((8, 128), jnp.int32),
          pltpu.VMEM((8, 128), jnp.int32),
          pltpu.VMEM((2, 128, 128), jnp.float32),
          pltpu.VMEM((2, 8, 8, 128), jnp.float32),
          pltpu.SemaphoreType.DMA,
          pltpu.SemaphoreType.DMA((2,)),
          pltpu.SemaphoreType.DMA((2,)),
      ],
  )
  def k2(s_hbm, i_hbm, o_hbm, idxb, pairb, gat, outb, isem, gsem, wsem):
    wid = lax.axis_index("s") * _NC + lax.axis_index("c")
    iota = lax.iota(jnp.int32, 16)
    tvecs = [iota + (t * 16) for t in range(8)]

    def gather_dma(r, b):
      return pltpu.make_async_copy(
          s_hbm.at[pairb.at[r]], gat.at[b], gsem.at[b])

    def out_wait(b):
      pltpu.make_async_copy(
          outb.at[b], o_hbm.at[0, :, 0], wsem.at[b]).wait()

    def body(m, carry):
      q = wid * per_w + m
      sg = q // NB
      bb = lax.rem(q, NB)

      pltpu.async_copy(
          i_hbm.at[pl.ds(sg * 8, 8), pl.ds(bb * 128, 128)], idxb,
          isem).wait()

      for rr in range(8):
        for g in range(8):
          ii = idxb[rr, pl.ds(g * 16, 16)]
          pairb[rr, pl.ds(g * 16, 16)] = lax.shift_right_logical(ii, 1)

      gather_dma(0, 0).start()
      gather_dma(1, 1).start()

      def rbody(r, rc):
        s_pos = sg * 8 + r

        for b in range(2):
          @pl.when(lax.rem(r, 2) == b)
          def _():
            gather_dma(r, b).wait()

            @pl.when(r >= 2)
            def _():
              out_wait(b)

            for t in range(8):
              par = lax.shift_left(
                  lax.bitwise_and(idxb[r, pl.ds(t * 16, 16)], 1), 6)

              @plsc.parallel_loop(0, 64, unroll=8)
              def _(d):
                cvec = par + d
                vals = plsc.load_gather(gat.at[b], [tvecs[t], cvec])
                outb[b, d // 8, lax.rem(d, 8), pl.ds(t * 16, 16)] = vals

            pltpu.async_copy(
                outb.at[b], o_hbm.at[s_pos, :, bb], wsem.at[b])

            @pl.when(r + 2 < 8)
            def _():
              gather_dma(r + 2, b).start()
        return rc

      lax.fori_loop(0, 8, rbody, 0)

      out_wait(0)
      out_wait(1)
      return carry

    lax.fori_loop(0, per_w, body, 0)

  return k2(scratch, idx_t)


def kernel(table, indices):
  table_t = table.T
  idx_t = indices.T.astype(jnp.int32)
  tail_packed = table[VFULL:].reshape(32, 128)

  scratch = _k1_relayout(table_t, tail_packed)
  out5 = _k2_gather(scratch, idx_t)
  return out5.transpose(2, 4, 0, 1, 3).reshape(BATCH, SEQ, D)


# speedup vs baseline: 1.8670x; 1.0311x over previous
import functools

import jax
import jax.numpy as jnp
from jax import lax
from jax.experimental import pallas as pl
from jax.experimental.pallas import tpu as pltpu
from jax.experimental.pallas import tpu_sc as plsc

_INFO = plsc.get_sparse_core_info()
_NC = _INFO.num_cores
_NS = _INFO.num_subcores
_NW = _NC * _NS

V = 1000000
D = 64
SEQ = 200
BATCH = 4096
NB = BATCH // 128
VFULL = (V // 128) * 128
NJ = VFULL // 128
SROWS = V // 2


def _k1_relayout(table_t, tail_packed):
  mesh = plsc.VectorSubcoreMesh(core_axis_name="c", subcore_axis_name="s")

  @functools.partial(
      pl.kernel,
      mesh=mesh,
      out_type=jax.ShapeDtypeStruct((SROWS, 128), jnp.float32),
      compiler_params=pltpu.CompilerParams(
          use_tc_tiling_on_sc=True, needs_layout_passes=False),
      scratch_types=[
          pltpu.VMEM((4, 64, 128), jnp.float32),
          pltpu.VMEM((2, 64, 128), jnp.float32),
          pltpu.VMEM((32, 128), jnp.float32),
          pltpu.SemaphoreType.DMA((4,)),
          pltpu.SemaphoreType.DMA((2,)),
          pltpu.SemaphoreType.DMA,
      ],
  )
  def k1(t_hbm, tail_hbm, s_hbm, ib, ob, tb, gsem, wsem, tsem):
    wid = lax.axis_index("s") * _NC + lax.axis_index("c")
    iota = lax.iota(jnp.int32, 16)
    dlanes = [iota + (g * 16) for g in range(4)]

    n_rounds = NJ // _NW + 1

    def in_dma(k, b):
      j = wid + k * _NW
      return pltpu.make_async_copy(
          t_hbm.at[:, pl.ds(j * 128, 128)], ib.at[b], gsem.at[b])

    def out_dma(k, w):
      j = wid + k * _NW
      return pltpu.make_async_copy(
          ob.at[w], s_hbm.at[pl.ds(j * 64, 64)], wsem.at[w])

    for b in range(4):
      in_dma(b, b).start()

    def body(k, carry):
      j = wid + k * _NW

      @pl.when(j < NJ)
      def _():
        for b in range(4):
          @pl.when(lax.rem(k, 4) == b)
          def _():
            w = b % 2
            in_dma(k, b).wait()

            @pl.when(k >= 2)
            def _():
              out_dma(k - 2, w).wait()

            @plsc.parallel_loop(0, 64, unroll=8)
            def _(jj):
              for p in range(2):
                vcol = jnp.full((16,), 2 * jj + p, jnp.int32)
                for g in range(4):
                  vals = plsc.load_gather(ib.at[b], [dlanes[g], vcol])
                  ob[w, jj, pl.ds(p * 64 + g * 16, 16)] = vals

            out_dma(k, w).start()

            @pl.when(j + 4 * _NW < NJ)
            def _():
              in_dma(k + 4, b).start()
      return carry

    lax.fori_loop(0, n_rounds, body, 0)

    pltpu.make_async_copy(
        ob.at[0], s_hbm.at[pl.ds(0, 64)], wsem.at[0]).wait()
    pltpu.make_async_copy(
        ob.at[1], s_hbm.at[pl.ds(0, 64)], wsem.at[1]).wait()

    @pl.when(wid == 0)
    def _():
      pltpu.async_copy(tail_hbm, tb, tsem).wait()
      pltpu.sync_copy(tb, s_hbm.at[pl.ds(NJ * 64, 32)])

  return k1(table_t, tail_packed)


def _k2_gather(scratch, idx_t):
  mesh = plsc.VectorSubcoreMesh(core_axis_name="c", subcore_axis_name="s")
  n_items = (SEQ // 8) * NB
  per_w = n_items // _NW

  @functools.partial(
      pl.kernel,
      mesh=mesh,
      out_type=jax.ShapeDtypeStruct((SEQ, 8, NB, 8, 128), jnp.float32),
      compiler_params=pltpu.CompilerParams(
          use_tc_tiling_on_sc=True, needs_layout_passes=False),
      scratch_types=[
          pltpu.VMEM((2, 8, 128), jnp.int32),
          pltpu.VMEM((2, 8, 128), jnp.int32),
          pltpu.VMEM((4, 128, 128), jnp.float32),
          pltpu.VMEM((2, 8, 8, 128), jnp.float32),
          pltpu.SemaphoreType.DMA((2,)),
          pltpu.SemaphoreType.DMA((4,)),
          pltpu.SemaphoreType.DMA((2,)),
      ],
  )
  def k2(s_hbm, i_hbm, o_hbm, idxb, pairb, gat, outb, isem, gsem, wsem):
    wid = lax.axis_index("s") * _NC + lax.axis_index("c")
    iota = lax.iota(jnp.int32, 16)
    tvecs = [iota + (t * 16) for t in range(8)]

    def idx_dma(m, mb):
      q = wid * per_w + m
      sg = q // NB
      bb = lax.rem(q, NB)
      return pltpu.make_async_copy(
          i_hbm.at[pl.ds(sg * 8, 8), pl.ds(bb * 128, 128)], idxb.at[mb],
          isem.at[mb])

    def gather_dma(mb, r, rb):
      return pltpu.make_async_copy(
          s_hbm.at[pairb.at[mb, r]], gat.at[rb], gsem.at[rb])

    def out_wait(w):
      pltpu.make_async_copy(
          outb.at[w], o_hbm.at[0, :, 0], wsem.at[w]).wait()

    idx_dma(0, 0).start()

    def body(m, carry):
      q = wid * per_w + m
      sg = q // NB
      bb = lax.rem(q, NB)

      for mb in range(2):
        @pl.when(lax.rem(m, 2) == mb)
        def _():
          idx_dma(m, mb).wait()

          @plsc.parallel_loop(0, 64, unroll=8)
          def _(i):
            rr = i // 8
            g = lax.rem(i, 8)
            ii = idxb[mb, rr, pl.ds(g * 16, 16)]
            pairb[mb, rr, pl.ds(g * 16, 16)] = lax.shift_right_logical(ii, 1)

          for rb in range(4):
            gather_dma(mb, rb, rb).start()

          @pl.when(m + 1 < per_w)
          def _():
            idx_dma(m + 1, 1 - mb).start()

          def rbody(r, rc):
            s_pos = sg * 8 + r

            for rb in range(4):
              @pl.when(lax.rem(r, 4) == rb)
              def _():
                w = rb % 2
                gather_dma(mb, r, rb).wait()

                @pl.when(r >= 2)
                def _():
                  out_wait(w)

                for t in range(8):
                  par = lax.shift_left(
                      lax.bitwise_and(idxb[mb, r, pl.ds(t * 16, 16)], 1), 6)

                  @plsc.parallel_loop(0, 64, unroll=8)
                  def _(d):
                    cvec = par + d
                    vals = plsc.load_gather(gat.at[rb], [tvecs[t], cvec])
                    outb[w, d // 8, lax.rem(d, 8), pl.ds(t * 16, 16)] = vals

                pltpu.async_copy(
                    outb.at[w], o_hbm.at[s_pos, :, bb], wsem.at[w])

                @pl.when(r + 4 < 8)
                def _():
                  gather_dma(mb, r + 4, rb).start()
            return rc

          lax.fori_loop(0, 8, rbody, 0)

      out_wait(0)
      out_wait(1)
      return carry

    lax.fori_loop(0, per_w, body, 0)

  return k2(scratch, idx_t)


def kernel(table, indices):
  table_t = table.T
  idx_t = indices.T.astype(jnp.int32)
  tail_packed = table[VFULL:].reshape(32, 128)

  scratch = _k1_relayout(table_t, tail_packed)
  out5 = _k2_gather(scratch, idx_t)
  return out5.transpose(2, 4, 0, 1, 3).reshape(BATCH, SEQ, D)


# speedup vs baseline: 1.9023x; 1.0189x over previous
import functools

import jax
import jax.numpy as jnp
from jax import lax
from jax.experimental import pallas as pl
from jax.experimental.pallas import tpu as pltpu
from jax.experimental.pallas import tpu_sc as plsc

_INFO = plsc.get_sparse_core_info()
_NC = _INFO.num_cores
_NS = _INFO.num_subcores
_NW = _NC * _NS

V = 1000000
D = 64
SEQ = 200
BATCH = 4096
NB = BATCH // 128
VFULL = (V // 128) * 128
NJ = VFULL // 128
SROWS = V // 2


def _k1_relayout(table_t, tail_packed):
  mesh = plsc.VectorSubcoreMesh(core_axis_name="c", subcore_axis_name="s")

  @functools.partial(
      pl.kernel,
      mesh=mesh,
      out_type=jax.ShapeDtypeStruct((SROWS, 128), jnp.float32),
      compiler_params=pltpu.CompilerParams(
          use_tc_tiling_on_sc=True, needs_layout_passes=False),
      scratch_types=[
          pltpu.VMEM((4, 64, 136), jnp.float32),
          pltpu.VMEM((2, 64, 128), jnp.float32),
          pltpu.VMEM((32, 128), jnp.float32),
          pltpu.SemaphoreType.DMA((4,)),
          pltpu.SemaphoreType.DMA((2,)),
          pltpu.SemaphoreType.DMA,
      ],
  )
  def k1(t_hbm, tail_hbm, s_hbm, ib, ob, tb, gsem, wsem, tsem):
    wid = lax.axis_index("s") * _NC + lax.axis_index("c")
    iota = lax.iota(jnp.int32, 16)
    dlanes = [iota + (g * 16) for g in range(4)]

    n_rounds = NJ // _NW + 1

    def in_dma(k, b):
      j = wid + k * _NW
      return pltpu.make_async_copy(
          t_hbm.at[:, pl.ds(j * 128, 128)], ib.at[b, :, pl.ds(0, 128)],
          gsem.at[b])

    def out_dma(k, w):
      j = wid + k * _NW
      return pltpu.make_async_copy(
          ob.at[w], s_hbm.at[pl.ds(j * 64, 64)], wsem.at[w])

    for b in range(4):
      in_dma(b, b).start()

    def body(k, carry):
      j = wid + k * _NW

      @pl.when(j < NJ)
      def _():
        for b in range(4):
          @pl.when(lax.rem(k, 4) == b)
          def _():
            w = b % 2
            in_dma(k, b).wait()

            @pl.when(k >= 2)
            def _():
              out_dma(k - 2, w).wait()

            @plsc.parallel_loop(0, 64, unroll=8)
            def _(jj):
              for p in range(2):
                vcol = jnp.full((16,), 2 * jj + p, jnp.int32)
                for g in range(4):
                  vals = plsc.load_gather(ib.at[b], [dlanes[g], vcol])
                  ob[w, jj, pl.ds(p * 64 + g * 16, 16)] = vals

            out_dma(k, w).start()

            @pl.when(j + 4 * _NW < NJ)
            def _():
              in_dma(k + 4, b).start()
      return carry

    lax.fori_loop(0, n_rounds, body, 0)

    pltpu.make_async_copy(
        ob.at[0], s_hbm.at[pl.ds(0, 64)], wsem.at[0]).wait()
    pltpu.make_async_copy(
        ob.at[1], s_hbm.at[pl.ds(0, 64)], wsem.at[1]).wait()

    @pl.when(wid == 0)
    def _():
      pltpu.async_copy(tail_hbm, tb, tsem).wait()
      pltpu.sync_copy(tb, s_hbm.at[pl.ds(NJ * 64, 32)])

  return k1(table_t, tail_packed)


def _k2_gather(scratch, idx_t):
  mesh = plsc.VectorSubcoreMesh(core_axis_name="c", subcore_axis_name="s")
  n_items = (SEQ // 8) * NB
  per_w = n_items // _NW

  @functools.partial(
      pl.kernel,
      mesh=mesh,
      out_type=jax.ShapeDtypeStruct((SEQ, 8, NB, 8, 128), jnp.float32),
      compiler_params=pltpu.CompilerParams(
          use_tc_tiling_on_sc=True, needs_layout_passes=False),
      scratch_types=[
          pltpu.VMEM((2, 8, 128), jnp.int32),
          pltpu.VMEM((2, 8, 128), jnp.int32),
          pltpu.VMEM((2, 128, 136), jnp.float32),
          pltpu.VMEM((2, 8, 8, 128), jnp.float32),
          pltpu.SemaphoreType.DMA((2,)),
          pltpu.SemaphoreType.DMA((2,)),
          pltpu.SemaphoreType.DMA((2,)),
      ],
  )
  def k2(s_hbm, i_hbm, o_hbm, idxb, pairb, gat, outb, isem, gsem, wsem):
    wid = lax.axis_index("s") * _NC + lax.axis_index("c")
    iota = lax.iota(jnp.int32, 16)
    tvecs = [iota + (t * 16) for t in range(8)]

    def idx_dma(m, mb):
      q = wid * per_w + m
      sg = q // NB
      bb = lax.rem(q, NB)
      return pltpu.make_async_copy(
          i_hbm.at[pl.ds(sg * 8, 8), pl.ds(bb * 128, 128)], idxb.at[mb],
          isem.at[mb])

    def gather_dma(mb, r, rb):
      return pltpu.make_async_copy(
          s_hbm.at[pairb.at[mb, r]], gat.at[rb, :, pl.ds(0, 128)],
          gsem.at[rb])

    def out_wait(w):
      pltpu.make_async_copy(
          outb.at[w], o_hbm.at[0, :, 0], wsem.at[w]).wait()

    idx_dma(0, 0).start()

    def body(m, carry):
      q = wid * per_w + m
      sg = q // NB
      bb = lax.rem(q, NB)

      for mb in range(2):
        @pl.when(lax.rem(m, 2) == mb)
        def _():
          idx_dma(m, mb).wait()

          @plsc.parallel_loop(0, 64, unroll=8)
          def _(i):
            rr = i // 8
            g = lax.rem(i, 8)
            ii = idxb[mb, rr, pl.ds(g * 16, 16)]
            pairb[mb, rr, pl.ds(g * 16, 16)] = lax.shift_right_logical(ii, 1)

          for rb in range(2):
            gather_dma(mb, rb, rb).start()

          @pl.when(m + 1 < per_w)
          def _():
            idx_dma(m + 1, 1 - mb).start()

          def rbody(r, rc):
            s_pos = sg * 8 + r

            for rb in range(2):
              @pl.when(lax.rem(r, 2) == rb)
              def _():
                w = rb
                gather_dma(mb, r, rb).wait()

                @pl.when(r >= 2)
                def _():
                  out_wait(w)

                for t in range(8):
                  par = lax.shift_left(
                      lax.bitwise_and(idxb[mb, r, pl.ds(t * 16, 16)], 1), 6)

                  @plsc.parallel_loop(0, 64, unroll=8)
                  def _(d):
                    cvec = par + d
                    vals = plsc.load_gather(gat.at[rb], [tvecs[t], cvec])
                    outb[w, d // 8, lax.rem(d, 8), pl.ds(t * 16, 16)] = vals

                pltpu.async_copy(
                    outb.at[w], o_hbm.at[s_pos, :, bb], wsem.at[w])

                @pl.when(r + 2 < 8)
                def _():
                  gather_dma(mb, r + 2, rb).start()
            return rc

          lax.fori_loop(0, 8, rbody, 0)

      out_wait(0)
      out_wait(1)
      return carry

    lax.fori_loop(0, per_w, body, 0)

  return k2(scratch, idx_t)


def kernel(table, indices):
  table_t = table.T
  idx_t = indices.T.astype(jnp.int32)
  tail_packed = table[VFULL:].reshape(32, 128)

  scratch = _k1_relayout(table_t, tail_packed)
  out5 = _k2_gather(scratch, idx_t)
  return out5.transpose(2, 4, 0, 1, 3).reshape(BATCH, SEQ, D)


# speedup vs baseline: 2.0933x; 1.1004x over previous
import functools

import jax
import jax.numpy as jnp
from jax import lax
from jax.experimental import pallas as pl
from jax.experimental.pallas import tpu as pltpu
from jax.experimental.pallas import tpu_sc as plsc

_INFO = plsc.get_sparse_core_info()
_NC = _INFO.num_cores
_NS = _INFO.num_subcores
_NW = _NC * _NS

V = 1000000
D = 64
SEQ = 200
BATCH = 4096
VFULL = (V // 128) * 128
NJ3 = VFULL // 384
SROWS = V // 2

CHUNK = 256
NBUF = 4


def _k1_relayout(table_t, tail_packed):
  mesh = plsc.VectorSubcoreMesh(core_axis_name="c", subcore_axis_name="s")

  @functools.partial(
      pl.kernel,
      mesh=mesh,
      out_type=jax.ShapeDtypeStruct((SROWS, 128), jnp.float32),
      compiler_params=pltpu.CompilerParams(
          use_tc_tiling_on_sc=True, needs_layout_passes=False),
      scratch_types=[
          pltpu.VMEM((2, 64, 384), jnp.float32),
          pltpu.VMEM((2, 192, 128), jnp.float32),
          pltpu.VMEM((32, 128), jnp.float32),
          pltpu.SemaphoreType.DMA((2,)),
          pltpu.SemaphoreType.DMA((2,)),
          pltpu.SemaphoreType.DMA,
      ],
  )
  def k1(t_hbm, tail_hbm, s_hbm, ib, ob, tb, gsem, wsem, tsem):
    wid = lax.axis_index("s") * _NC + lax.axis_index("c")
    iota = lax.iota(jnp.int32, 16)
    dlanes = [iota + (g * 16) for g in range(4)]

    n_rounds = NJ3 // _NW + 1

    def in_dma(k, b):
      j = wid + k * _NW
      return pltpu.make_async_copy(
          t_hbm.at[:, pl.ds(j * 384, 384)], ib.at[b], gsem.at[b])

    def out_dma(k, b):
      j = wid + k * _NW
      return pltpu.make_async_copy(
          ob.at[b], s_hbm.at[pl.ds(j * 192, 192)], wsem.at[b])

    in_dma(0, 0).start()
    in_dma(1, 1).start()

    def body(k, carry):
      j = wid + k * _NW

      @pl.when(j < NJ3)
      def _():
        for b in range(2):
          @pl.when(lax.rem(k, 2) == b)
          def _():
            in_dma(k, b).wait()

            @pl.when(k >= 2)
            def _():
              out_dma(k - 2, b).wait()

            @plsc.parallel_loop(0, 192, unroll=8)
            def _(rr):
              for p in range(2):
                vcol = jnp.full((16,), 2 * rr + p, jnp.int32)
                for g in range(4):
                  vals = plsc.load_gather(ib.at[b], [dlanes[g], vcol])
                  ob[b, rr, pl.ds(p * 64 + g * 16, 16)] = vals

            out_dma(k, b).start()

            @pl.when(j + 2 * _NW < NJ3)
            def _():
              in_dma(k + 2, b).start()
      return carry

    lax.fori_loop(0, n_rounds, body, 0)

    pltpu.make_async_copy(
        ob.at[0], s_hbm.at[pl.ds(0, 192)], wsem.at[0]).wait()
    pltpu.make_async_copy(
        ob.at[1], s_hbm.at[pl.ds(0, 192)], wsem.at[1]).wait()

    @pl.when(wid == 0)
    def _():
      pltpu.async_copy(tail_hbm, tb, tsem).wait()
      pltpu.sync_copy(tb, s_hbm.at[pl.ds(SROWS - 32, 32)])

  return k1(table_t, tail_packed)


def _gather_rows(table, idx2, n_workers, n_chunks, d):
  npad = n_workers * n_chunks * CHUNK
  per_w = n_chunks * CHUNK
  mesh = plsc.VectorSubcoreMesh(core_axis_name="c", subcore_axis_name="s")
  info = plsc.get_sparse_core_info()

  @functools.partial(
      pl.kernel,
      mesh=mesh,
      out_type=jax.ShapeDtypeStruct((npad, d), jnp.float32),
      compiler_params=pltpu.CompilerParams(use_tc_tiling_on_sc=False),
      scratch_types=[
          pltpu.VMEM((per_w,), jnp.int32),
          pltpu.VMEM((NBUF, CHUNK, d), jnp.float32),
          pltpu.SemaphoreType.DMA((NBUF,)),
          pltpu.SemaphoreType.DMA((NBUF,)),
      ],
  )
  def k(table_hbm, idx_hbm, out_hbm, idx_v, rows_v, gsem, wsem):
    wid = lax.axis_index("s") * info.num_cores + lax.axis_index("c")
    pltpu.sync_copy(idx_hbm.at[wid], idx_v)
    base = wid * per_w

    def gfire(j, b):
      pltpu.async_copy(
          table_hbm.at[idx_v.at[pl.ds(j * CHUNK, CHUNK)]],
          rows_v.at[b], gsem.at[b])

    def gwait(j, b):
      pltpu.make_async_copy(
          table_hbm.at[idx_v.at[pl.ds(j * CHUNK, CHUNK)]],
          rows_v.at[b], gsem.at[b]).wait()

    def wfire(j, b):
      pltpu.async_copy(
          rows_v.at[b], out_hbm.at[pl.ds(base + j * CHUNK, CHUNK)], wsem.at[b])

    def wwait(j, b):
      pltpu.make_async_copy(
          rows_v.at[b], out_hbm.at[pl.ds(base + j * CHUNK, CHUNK)],
          wsem.at[b]).wait()

    for b in range(NBUF):
      gfire(b, b)

    def body(i, carry):
      for b in range(NBUF):
        j = i * NBUF + b
        gwait(j, b)
        wfire(j, b)
        wwait(j, b)
        gfire(j + NBUF, b)
      return carry

    lax.fori_loop(0, n_chunks // NBUF - 1, body, 0)

    for b in range(NBUF):
      j = n_chunks - NBUF + b
      gwait(j, b)
      wfire(j, b)
      wwait(j, b)

  return k(table, idx2)


def kernel(table, indices):
  b, s = indices.shape
  v, d = table.shape
  n = b * s

  table_t = table.T
  tail_packed = table[VFULL:].reshape(32, 128)
  scratch = _k1_relayout(table_t, tail_packed)
  table_lin = scratch.reshape(v, d)

  idx = indices.T.reshape(n).astype(jnp.int32)
  n_chunks = n // (_NW * CHUNK)
  idx2 = idx.reshape(_NW, n_chunks * CHUNK)

  out = _gather_rows(table_lin, idx2, _NW, n_chunks, d)
  return out.reshape(s, b, d).transpose(1, 0, 2)
